# baseline (speedup 1.0000x reference)
import functools

import jax
import jax.numpy as jnp
from jax.experimental import pallas as pl
from jax.experimental.pallas import tpu as pltpu

_TN = 512
_MC = 128
_K = 8


def _chamfer_body(x1_ref, x2_ref, dist_ref, idx_ref,
                  bneg_s, *, M, NB):
    f32 = jnp.float32
    bf16 = jnp.bfloat16
    step = pl.program_id(0)

    @pl.when(step % NB == 0)
    def _build_scratch():
        b = x2_ref[0]
        bx, by, bz = b[0:1, :], b[1:2, :], b[2:3, :]
        y2 = bx * bx + by * by + bz * bz
        y2a = y2.astype(bf16)
        r1 = y2 - y2a.astype(f32)
        y2b = r1.astype(bf16)
        r2 = r1 - y2b.astype(f32)
        y2c = r2.astype(bf16)
        bneg_s[0:3, :] = b.astype(bf16) * jnp.asarray(-2.0, bf16)
        bneg_s[3:4, :] = y2a
        bneg_s[4:5, :] = y2b
        bneg_s[5:6, :] = y2c
        bneg_s[6:_K, :] = jnp.ones((_K - 6, M), bf16)

    a = x1_ref[0]
    ax, ay, az = a[0:1, :], a[1:2, :], a[2:3, :]
    x2 = ax * ax + ay * ay + az * az
    x2a = x2.astype(bf16)
    x2b = (x2 - x2a.astype(f32)).astype(bf16)
    ones = jnp.ones((_K - 5, _TN), bf16)
    a8 = jnp.concatenate([a.astype(bf16), ones, x2a, x2b], axis=0)

    dims = (((0,), (0,)), ((), ()))
    i32 = jnp.int32
    nchunks = M // _MC
    assert nchunks <= 64

    runkey = jnp.full((_MC, _TN), jnp.iinfo(i32).max, i32)
    for c in range(nchunks):
        bneg = bneg_s[:, c * _MC:(c + 1) * _MC]
        d = jax.lax.dot_general(bneg, a8, dims,
                                preferred_element_type=f32)
        key = (jax.lax.bitcast_convert_type(d, i32) & ~63) | c
        runkey = jnp.minimum(runkey, key)

    kmin = jnp.min(runkey, axis=0, keepdims=True)
    rowiota = jax.lax.broadcasted_iota(i32, (_MC, _TN), 0)
    cand = jnp.where(runkey == kmin, rowiota, M)
    rmin = jnp.min(cand, axis=0, keepdims=True)

    dist_ref[0] = jax.lax.bitcast_convert_type(kmin & ~63, f32)
    idx_ref[0] = (kmin & 63) * _MC + rmin


@jax.jit
def kernel(xyz1, xyz2):
    xyz1 = xyz1.astype(jnp.float32)
    xyz2 = xyz2.astype(jnp.float32)
    B, N, _ = xyz1.shape
    M = xyz2.shape[1]
    NB = N // _TN

    x1t = jnp.transpose(xyz1, (0, 2, 1))
    x2t = jnp.transpose(xyz2, (0, 2, 1))

    grid = (B * NB,)
    dist, idx = pl.pallas_call(
        functools.partial(_chamfer_body, M=M, NB=NB),
        grid=grid,
        in_specs=[
            pl.BlockSpec((1, 3, _TN), lambda g: (g // NB, 0, g % NB)),
            pl.BlockSpec((1, 3, M), lambda g: (g // NB, 0, 0)),
        ],
        out_specs=[
            pl.BlockSpec((1, 1, _TN), lambda g: (g, 0, 0)),
            pl.BlockSpec((1, 1, _TN), lambda g: (g, 0, 0)),
        ],
        out_shape=[
            jax.ShapeDtypeStruct((B * NB, 1, _TN), jnp.float32),
            jax.ShapeDtypeStruct((B * NB, 1, _TN), jnp.int32),
        ],
        scratch_shapes=[
            pltpu.VMEM((_K, M), jnp.bfloat16),
        ],
    )(x1t, x2t)

    return dist.reshape(B, N), idx.reshape(B, N)


# speedup vs baseline: 1.0150x; 1.0150x over previous
import functools

import jax
import jax.numpy as jnp
from jax.experimental import pallas as pl
from jax.experimental.pallas import tpu as pltpu

_TN = 512
_MC = 128
_K = 8


def _chamfer_body(x1_ref, x2_ref, dist_ref, idx_ref,
                  bneg_s, *, M, NB):
    f32 = jnp.float32
    bf16 = jnp.bfloat16
    step = pl.program_id(0)

    @pl.when(step % NB == 0)
    def _build_scratch():
        b = x2_ref[0]
        bx, by, bz = b[0:1, :], b[1:2, :], b[2:3, :]
        y2 = bx * bx + by * by + bz * bz
        y2a = y2.astype(bf16)
        r1 = y2 - y2a.astype(f32)
        y2b = r1.astype(bf16)
        r2 = r1 - y2b.astype(f32)
        y2c = r2.astype(bf16)
        bneg_s[0:3, :] = b.astype(bf16) * jnp.asarray(-2.0, bf16)
        bneg_s[3:4, :] = y2a
        bneg_s[4:5, :] = y2b
        bneg_s[5:6, :] = y2c
        bneg_s[6:_K, :] = jnp.ones((_K - 6, M), bf16)

    a = x1_ref[0]
    ax, ay, az = a[0:1, :], a[1:2, :], a[2:3, :]
    x2 = ax * ax + ay * ay + az * az
    x2a = x2.astype(bf16)
    x2b = (x2 - x2a.astype(f32)).astype(bf16)
    ones = jnp.ones((_K - 5, _TN), bf16)
    a8 = jnp.concatenate([a.astype(bf16), ones, x2a, x2b], axis=0)

    dims = (((0,), (0,)), ((), ()))
    i32 = jnp.int32
    nchunks = M // _MC
    assert nchunks <= 64

    runkey = jnp.full((_MC, _TN), jnp.inf, f32)
    for c in range(nchunks):
        bneg = bneg_s[:, c * _MC:(c + 1) * _MC]
        d = jax.lax.dot_general(bneg, a8, dims,
                                preferred_element_type=f32)
        key = jax.lax.bitcast_convert_type(
            (jax.lax.bitcast_convert_type(d, i32) & ~63) | c, f32)
        runkey = jnp.minimum(runkey, key)

    kmin = jnp.min(runkey, axis=0, keepdims=True)
    kbits = jax.lax.bitcast_convert_type(kmin, i32)
    rowiota = jax.lax.broadcasted_iota(i32, (_MC, _TN), 0)
    cand = jnp.where(runkey == kmin, rowiota, M)
    rmin = jnp.min(cand, axis=0, keepdims=True)

    dist_ref[0] = jax.lax.bitcast_convert_type(kbits & ~63, f32)
    idx_ref[0] = (kbits & 63) * _MC + rmin


@jax.jit
def kernel(xyz1, xyz2):
    xyz1 = xyz1.astype(jnp.float32)
    xyz2 = xyz2.astype(jnp.float32)
    B, N, _ = xyz1.shape
    M = xyz2.shape[1]
    NB = N // _TN

    x1t = jnp.transpose(xyz1, (0, 2, 1))
    x2t = jnp.transpose(xyz2, (0, 2, 1))

    grid = (B * NB,)
    dist, idx = pl.pallas_call(
        functools.partial(_chamfer_body, M=M, NB=NB),
        grid=grid,
        in_specs=[
            pl.BlockSpec((1, 3, _TN), lambda g: (g // NB, 0, g % NB)),
            pl.BlockSpec((1, 3, M), lambda g: (g // NB, 0, 0)),
        ],
        out_specs=[
            pl.BlockSpec((1, 1, _TN), lambda g: (g, 0, 0)),
            pl.BlockSpec((1, 1, _TN), lambda g: (g, 0, 0)),
        ],
        out_shape=[
            jax.ShapeDtypeStruct((B * NB, 1, _TN), jnp.float32),
            jax.ShapeDtypeStruct((B * NB, 1, _TN), jnp.int32),
        ],
        scratch_shapes=[
            pltpu.VMEM((_K, M), jnp.bfloat16),
        ],
    )(x1t, x2t)

    return dist.reshape(B, N), idx.reshape(B, N)


# speedup vs baseline: 1.0237x; 1.0086x over previous
import functools

import jax
import jax.numpy as jnp
from jax.experimental import pallas as pl
from jax.experimental.pallas import tpu as pltpu

_TN = 512
_MC = 64
_K = 8


def _chamfer_body(x1_ref, x2_ref, dist_ref, idx_ref,
                  bneg_s, bneg_s2, *, M, NB):
    f32 = jnp.float32
    bf16 = jnp.bfloat16
    step = pl.program_id(0)

    @pl.when(step % NB == 0)
    def _build_scratch():
        b = x2_ref[0]
        bx, by, bz = b[0:1, :], b[1:2, :], b[2:3, :]
        y2 = bx * bx + by * by + bz * bz
        y2a = y2.astype(bf16)
        r1 = y2 - y2a.astype(f32)
        y2b = r1.astype(bf16)
        r2 = r1 - y2b.astype(f32)
        y2c = r2.astype(bf16)
        bneg3 = b.astype(bf16) * jnp.asarray(-2.0, bf16)
        bneg_s[0:3, :] = bneg3
        bneg_s[3:4, :] = y2a
        bneg_s[4:5, :] = y2b
        bneg_s[5:6, :] = y2c
        bneg_s[6:_K, :] = jnp.ones((_K - 6, M), bf16)
        bneg_s2[0:3, 0:M - _MC] = bneg3[:, _MC:]
        bneg_s2[3:4, 0:M - _MC] = y2a[:, _MC:]
        bneg_s2[4:5, 0:M - _MC] = y2b[:, _MC:]
        bneg_s2[5:6, 0:M - _MC] = y2c[:, _MC:]
        bneg_s2[6:_K, :] = jnp.ones((_K - 6, M), bf16)

    a = x1_ref[0]
    ax, ay, az = a[0:1, :], a[1:2, :], a[2:3, :]
    x2 = ax * ax + ay * ay + az * az
    x2a = x2.astype(bf16)
    x2b = (x2 - x2a.astype(f32)).astype(bf16)
    ones = jnp.ones((_K - 5, _TN), bf16)
    a8 = jnp.concatenate([a.astype(bf16), ones, x2a, x2b], axis=0)

    dims = (((0,), (0,)), ((), ()))

    runmin = jnp.full((_MC, _TN), jnp.inf, f32)
    runc = jnp.zeros((_MC, _TN), jnp.int32)
    for c in range(M // _MC):
        if c % 2 == 0:
            bneg = bneg_s[:, c * _MC:(c + 1) * _MC]
        else:
            bneg = bneg_s2[:, (c - 1) * _MC:c * _MC]
        d = jax.lax.dot_general(bneg, a8, dims,
                                preferred_element_type=f32)
        mask = d < runmin
        runmin = jnp.where(mask, d, runmin)
        runc = jnp.where(mask, c, runc)

    dmin = jnp.min(runmin, axis=0, keepdims=True)
    rowiota = jax.lax.broadcasted_iota(jnp.int32, (_MC, _TN), 0)
    cand = jnp.where(runmin == dmin, runc * _MC + rowiota, M)
    imin = jnp.min(cand, axis=0, keepdims=True)

    dist_ref[0] = dmin
    idx_ref[0] = imin


@jax.jit
def kernel(xyz1, xyz2):
    xyz1 = xyz1.astype(jnp.float32)
    xyz2 = xyz2.astype(jnp.float32)
    B, N, _ = xyz1.shape
    M = xyz2.shape[1]
    NB = N // _TN

    x1t = jnp.transpose(xyz1, (0, 2, 1))
    x2t = jnp.transpose(xyz2, (0, 2, 1))

    grid = (B * NB,)
    dist, idx = pl.pallas_call(
        functools.partial(_chamfer_body, M=M, NB=NB),
        grid=grid,
        in_specs=[
            pl.BlockSpec((1, 3, _TN), lambda g: (g // NB, 0, g % NB)),
            pl.BlockSpec((1, 3, M), lambda g: (g // NB, 0, 0)),
        ],
        out_specs=[
            pl.BlockSpec((1, 1, _TN), lambda g: (g, 0, 0)),
            pl.BlockSpec((1, 1, _TN), lambda g: (g, 0, 0)),
        ],
        out_shape=[
            jax.ShapeDtypeStruct((B * NB, 1, _TN), jnp.float32),
            jax.ShapeDtypeStruct((B * NB, 1, _TN), jnp.int32),
        ],
        scratch_shapes=[
            pltpu.VMEM((_K, M), jnp.bfloat16),
            pltpu.VMEM((_K, M), jnp.bfloat16),
        ],
    )(x1t, x2t)

    return dist.reshape(B, N), idx.reshape(B, N)


# speedup vs baseline: 1.0293x; 1.0055x over previous
import functools

import jax
import jax.numpy as jnp
from jax.experimental import pallas as pl
from jax.experimental.pallas import tpu as pltpu

_TN = 512
_MC = 64
_K = 8


def _chamfer_body(x1_ref, x2_ref, dist_ref, idx_ref,
                  bneg_s, bneg_s2, *, M, NB):
    f32 = jnp.float32
    bf16 = jnp.bfloat16
    t = pl.program_id(1)

    @pl.when(t == 0)
    def _build_scratch():
        b = x2_ref[0]
        bx, by, bz = b[0:1, :], b[1:2, :], b[2:3, :]
        y2 = bx * bx + by * by + bz * bz
        y2a = y2.astype(bf16)
        r1 = y2 - y2a.astype(f32)
        y2b = r1.astype(bf16)
        r2 = r1 - y2b.astype(f32)
        y2c = r2.astype(bf16)
        bneg3 = b.astype(bf16) * jnp.asarray(-2.0, bf16)
        bneg_s[0:3, :] = bneg3
        bneg_s[3:4, :] = y2a
        bneg_s[4:5, :] = y2b
        bneg_s[5:6, :] = y2c
        bneg_s[6:_K, :] = jnp.ones((_K - 6, M), bf16)
        bneg_s2[0:3, 0:M - _MC] = bneg3[:, _MC:]
        bneg_s2[3:4, 0:M - _MC] = y2a[:, _MC:]
        bneg_s2[4:5, 0:M - _MC] = y2b[:, _MC:]
        bneg_s2[5:6, 0:M - _MC] = y2c[:, _MC:]
        bneg_s2[6:_K, :] = jnp.ones((_K - 6, M), bf16)

    a = x1_ref[0]
    ax, ay, az = a[0:1, :], a[1:2, :], a[2:3, :]
    x2 = ax * ax + ay * ay + az * az
    x2a = x2.astype(bf16)
    x2b = (x2 - x2a.astype(f32)).astype(bf16)
    ones = jnp.ones((_K - 5, _TN), bf16)
    a8 = jnp.concatenate([a.astype(bf16), ones, x2a, x2b], axis=0)

    dims = (((0,), (0,)), ((), ()))

    runmin = jnp.full((_MC, _TN), jnp.inf, f32)
    runc = jnp.zeros((_MC, _TN), jnp.int32)
    for c in range(M // _MC):
        if c % 2 == 0:
            bneg = bneg_s[:, c * _MC:(c + 1) * _MC]
        else:
            bneg = bneg_s2[:, (c - 1) * _MC:c * _MC]
        d = jax.lax.dot_general(bneg, a8, dims,
                                preferred_element_type=f32)
        mask = d < runmin
        runmin = jnp.where(mask, d, runmin)
        runc = jnp.where(mask, c, runc)

    dmin = jnp.min(runmin, axis=0, keepdims=True)
    rowiota = jax.lax.broadcasted_iota(jnp.int32, (_MC, _TN), 0)
    cand = jnp.where(runmin == dmin, runc * _MC + rowiota, M)
    imin = jnp.min(cand, axis=0, keepdims=True)

    dist_ref[0] = dmin
    idx_ref[0] = imin


@jax.jit
def kernel(xyz1, xyz2):
    xyz1 = xyz1.astype(jnp.float32)
    xyz2 = xyz2.astype(jnp.float32)
    B, N, _ = xyz1.shape
    M = xyz2.shape[1]
    NB = N // _TN

    x1t = jnp.transpose(xyz1, (0, 2, 1))
    x2t = jnp.transpose(xyz2, (0, 2, 1))

    grid = (B, NB)
    dist, idx = pl.pallas_call(
        functools.partial(_chamfer_body, M=M, NB=NB),
        grid=grid,
        in_specs=[
            pl.BlockSpec((1, 3, _TN), lambda b, t: (b, 0, t)),
            pl.BlockSpec((1, 3, M), lambda b, t: (b, 0, 0)),
        ],
        out_specs=[
            pl.BlockSpec((1, 1, _TN), lambda b, t: (b * NB + t, 0, 0)),
            pl.BlockSpec((1, 1, _TN), lambda b, t: (b * NB + t, 0, 0)),
        ],
        out_shape=[
            jax.ShapeDtypeStruct((B * NB, 1, _TN), jnp.float32),
            jax.ShapeDtypeStruct((B * NB, 1, _TN), jnp.int32),
        ],
        scratch_shapes=[
            pltpu.VMEM((_K, M), jnp.bfloat16),
            pltpu.VMEM((_K, M), jnp.bfloat16),
        ],
        compiler_params=pltpu.CompilerParams(
            dimension_semantics=("parallel", "arbitrary")),
    )(x1t, x2t)

    return dist.reshape(B, N), idx.reshape(B, N)


# speedup vs baseline: 1.1225x; 1.0905x over previous
import functools

import jax
import jax.numpy as jnp
from jax.experimental import pallas as pl
from jax.experimental.pallas import tpu as pltpu

_TN = 1024
_MC = 64
_K = 8


def _chamfer_body(x1_ref, x2_ref, dist_ref, idx_ref,
                  bneg_s, bneg_s2, *, M, NB):
    f32 = jnp.float32
    bf16 = jnp.bfloat16
    t = pl.program_id(1)

    @pl.when(t == 0)
    def _build_scratch():
        b = x2_ref[0]
        bx, by, bz = b[0:1, :], b[1:2, :], b[2:3, :]
        y2 = bx * bx + by * by + bz * bz
        y2a = y2.astype(bf16)
        r1 = y2 - y2a.astype(f32)
        y2b = r1.astype(bf16)
        r2 = r1 - y2b.astype(f32)
        y2c = r2.astype(bf16)
        bneg3 = b.astype(bf16) * jnp.asarray(-2.0, bf16)
        bneg_s[0:3, :] = bneg3
        bneg_s[3:4, :] = y2a
        bneg_s[4:5, :] = y2b
        bneg_s[5:6, :] = y2c
        bneg_s[6:_K, :] = jnp.ones((_K - 6, M), bf16)
        bneg_s2[0:3, 0:M - _MC] = bneg3[:, _MC:]
        bneg_s2[3:4, 0:M - _MC] = y2a[:, _MC:]
        bneg_s2[4:5, 0:M - _MC] = y2b[:, _MC:]
        bneg_s2[5:6, 0:M - _MC] = y2c[:, _MC:]
        bneg_s2[6:_K, :] = jnp.ones((_K - 6, M), bf16)

    a = x1_ref[0]
    ax, ay, az = a[0:1, :], a[1:2, :], a[2:3, :]
    x2 = ax * ax + ay * ay + az * az
    x2a = x2.astype(bf16)
    x2b = (x2 - x2a.astype(f32)).astype(bf16)
    ones = jnp.ones((_K - 5, _TN), bf16)
    a8 = jnp.concatenate([a.astype(bf16), ones, x2a, x2b], axis=0)

    dims = (((0,), (0,)), ((), ()))

    runmin = jnp.full((_MC, _TN), jnp.inf, f32)
    runc = jnp.zeros((_MC, _TN), jnp.int32)
    for c in range(M // _MC):
        if c % 2 == 0:
            bneg = bneg_s[:, c * _MC:(c + 1) * _MC]
        else:
            bneg = bneg_s2[:, (c - 1) * _MC:c * _MC]
        d = jax.lax.dot_general(bneg, a8, dims,
                                preferred_element_type=f32)
        mask = d < runmin
        runmin = jnp.where(mask, d, runmin)
        runc = jnp.where(mask, c, runc)

    dmin = jnp.min(runmin, axis=0, keepdims=True)
    rowiota = jax.lax.broadcasted_iota(jnp.int32, (_MC, _TN), 0)
    cand = jnp.where(runmin == dmin, runc * _MC + rowiota, M)
    imin = jnp.min(cand, axis=0, keepdims=True)

    dist_ref[0] = dmin
    idx_ref[0] = imin


@jax.jit
def kernel(xyz1, xyz2):
    xyz1 = xyz1.astype(jnp.float32)
    xyz2 = xyz2.astype(jnp.float32)
    B, N, _ = xyz1.shape
    M = xyz2.shape[1]
    NB = N // _TN

    x1t = jnp.transpose(xyz1, (0, 2, 1))
    x2t = jnp.transpose(xyz2, (0, 2, 1))

    grid = (B, NB)
    dist, idx = pl.pallas_call(
        functools.partial(_chamfer_body, M=M, NB=NB),
        grid=grid,
        in_specs=[
            pl.BlockSpec((1, 3, _TN), lambda b, t: (b, 0, t)),
            pl.BlockSpec((1, 3, M), lambda b, t: (b, 0, 0)),
        ],
        out_specs=[
            pl.BlockSpec((1, 1, _TN), lambda b, t: (b * NB + t, 0, 0)),
            pl.BlockSpec((1, 1, _TN), lambda b, t: (b * NB + t, 0, 0)),
        ],
        out_shape=[
            jax.ShapeDtypeStruct((B * NB, 1, _TN), jnp.float32),
            jax.ShapeDtypeStruct((B * NB, 1, _TN), jnp.int32),
        ],
        scratch_shapes=[
            pltpu.VMEM((_K, M), jnp.bfloat16),
            pltpu.VMEM((_K, M), jnp.bfloat16),
        ],
        compiler_params=pltpu.CompilerParams(
            dimension_semantics=("parallel", "arbitrary")),
    )(x1t, x2t)

    return dist.reshape(B, N), idx.reshape(B, N)
